# baseline (speedup 1.0000x reference)
import functools

import jax
import jax.numpy as jnp
from jax import lax
from jax.experimental import pallas as pl
from jax.experimental.pallas import tpu as pltpu
from jax.experimental.pallas import tpu_sc as plsc

H = 2048
W = 2048
C = 4
PS = 256
NPX = W // PS
OUT_H = 1024
OUT_W = 1024
NPIX = OUT_H * OUT_W

NC = 2
NS = 16
NW = NC * NS

P_PER_W = NPIX // NW
B = 1024
NCHUNK = P_PER_W // B
GB = 128
K = B // GB
VL = 16
NV = B // VL


def _body(table, texc, out, texc_v, i00, i10, i01, i11, fx_v, fy_v,
          r00, r10, r01, r11, out_v, sem):
    wid = lax.axis_index("s") * NC + lax.axis_index("c")
    lanes = lax.iota(jnp.int32, VL)

    def zero_body(j, carry):
        z = lanes * 0
        for ref in (i00, i10, i01, i11):
            ref[pl.ds(j * VL, VL)] = z
        return carry

    lax.fori_loop(0, 2 * B // VL, zero_body, 0, unroll=False)

    def chunk_body(chunk, carry):
        base_pix = wid * P_PER_W + chunk * B
        pltpu.sync_copy(texc.at[pl.ds(base_pix * 2, 2 * B)], texc_v)

        def idx_body(j, carry):
            b16 = j * VL
            t2 = lanes * 2 + b16 * 2
            u = plsc.load_gather(texc_v, [t2])
            v = plsc.load_gather(texc_v, [t2 + 1])
            uu = u * float(W) - 0.5
            vv = v * float(H) - 0.5
            iut = uu.astype(jnp.int32)
            ivt = vv.astype(jnp.int32)
            ix0 = iut - (uu < iut.astype(jnp.float32)).astype(jnp.int32)
            iy0 = ivt - (vv < ivt.astype(jnp.float32)).astype(jnp.int32)
            fx = uu - ix0.astype(jnp.float32)
            fy = vv - iy0.astype(jnp.float32)
            ixw0 = ix0 & (W - 1)
            ixw1 = (ix0 + 1) & (W - 1)
            iyw0 = iy0 & (H - 1)
            iyw1 = (iy0 + 1) & (H - 1)
            xp0 = ((ixw0 >> 8) << 16) | (ixw0 & (PS - 1))
            xp1 = ((ixw1 >> 8) << 16) | (ixw1 & (PS - 1))
            yp0 = ((iyw0 >> 8) << 19) | ((iyw0 & (PS - 1)) << 8)
            yp1 = ((iyw1 >> 8) << 19) | ((iyw1 & (PS - 1)) << 8)
            slot = (lanes + b16) * 2
            plsc.store_scatter(i00, [slot], (yp0 | xp0) * 2)
            plsc.store_scatter(i10, [slot], (yp0 | xp1) * 2)
            plsc.store_scatter(i01, [slot], (yp1 | xp0) * 2)
            plsc.store_scatter(i11, [slot], (yp1 | xp1) * 2)
            fx_v[pl.ds(b16, VL)] = fx
            fy_v[pl.ds(b16, VL)] = fy
            return carry

        lax.fori_loop(0, NV, idx_body, 0, unroll=False)

        def fire(k, carry):
            for idx, rows in ((i00, r00), (i10, r10), (i01, r01), (i11, r11)):
                pltpu.make_async_copy(
                    table.at[idx.at[pl.ds(k * 2 * GB, 2 * GB)]],
                    rows.at[pl.ds(k * 2 * GB, 2 * GB)], sem).start()
            return carry

        lax.fori_loop(0, K, fire, 0, unroll=False)

        def drain(k, carry):
            for idx, rows in ((i00, r00), (i10, r10), (i01, r01), (i11, r11)):
                pltpu.make_async_copy(
                    table.at[idx.at[pl.ds(k * 2 * GB, 2 * GB)]],
                    rows.at[pl.ds(k * 2 * GB, 2 * GB)], sem).wait()
            return carry

        lax.fori_loop(0, K, drain, 0, unroll=False)

        def blend_body(j, carry):
            b16 = j * VL
            pix = lanes + b16
            row = pix + (b16 & ~(GB - 1))
            fx = fx_v[pl.ds(b16, VL)]
            fy = fy_v[pl.ds(b16, VL)]
            gx = 1.0 - fx
            gy = 1.0 - fy
            w00 = gx * gy
            w10 = fx * gy
            w01 = gx * fy
            w11 = fx * fy
            for c in range(C):
                cc = lanes * 0 + c
                c00 = plsc.load_gather(r00, [row, cc])
                c10 = plsc.load_gather(r10, [row, cc])
                c01 = plsc.load_gather(r01, [row, cc])
                c11 = plsc.load_gather(r11, [row, cc])
                o = c00 * w00 + c10 * w10 + c01 * w01 + c11 * w11
                plsc.store_scatter(out_v, [pix, cc], o)
            return carry

        lax.fori_loop(0, NV, blend_body, 0, unroll=False)

        pltpu.sync_copy(out_v, out.at[pl.ds(base_pix, B)])
        return carry

    lax.fori_loop(0, NCHUNK, chunk_body, 0, unroll=False)


@jax.jit
def _sample(table, texc_flat):
    mesh = plsc.VectorSubcoreMesh(core_axis_name="c", subcore_axis_name="s")
    f = pl.kernel(
        _body,
        out_type=jax.ShapeDtypeStruct((NPIX, C), jnp.float32),
        mesh=mesh,
        compiler_params=pltpu.CompilerParams(
            needs_layout_passes=False, use_tc_tiling_on_sc=False),
        scratch_types=[
            pltpu.VMEM((2 * B,), jnp.float32),
            pltpu.VMEM((2 * B,), jnp.int32),
            pltpu.VMEM((2 * B,), jnp.int32),
            pltpu.VMEM((2 * B,), jnp.int32),
            pltpu.VMEM((2 * B,), jnp.int32),
            pltpu.VMEM((B,), jnp.float32),
            pltpu.VMEM((B,), jnp.float32),
            pltpu.VMEM((2 * B, C), jnp.float32),
            pltpu.VMEM((2 * B, C), jnp.float32),
            pltpu.VMEM((2 * B, C), jnp.float32),
            pltpu.VMEM((2 * B, C), jnp.float32),
            pltpu.VMEM((B, C), jnp.float32),
            pltpu.SemaphoreType.DMA,
        ],
    )
    return f(table, texc_flat)


def kernel(pages, texc):
    table = pages.reshape(H * W, C)
    texc_flat = texc.reshape(-1)
    out = _sample(table, texc_flat)
    return out.reshape(1, OUT_H, OUT_W, C)


# speedup vs baseline: 1.0008x; 1.0008x over previous
import functools

import jax
import jax.numpy as jnp
from jax import lax
from jax.experimental import pallas as pl
from jax.experimental.pallas import tpu as pltpu
from jax.experimental.pallas import tpu_sc as plsc

H = 2048
W = 2048
C = 4
PS = 256
NPX = W // PS
OUT_H = 1024
OUT_W = 1024
NPIX = OUT_H * OUT_W

NC = 2
NS = 16
NW = NC * NS

P_PER_W = NPIX // NW
B = 1024
NCHUNK = P_PER_W // B
GB = 512
K = B // GB
VL = 16
NV = B // VL


def _body(table, texc, out, texc_v, i00, i10, i01, i11, fx_v, fy_v,
          r00, r10, r01, r11, out_v, sem):
    wid = lax.axis_index("s") * NC + lax.axis_index("c")
    lanes = lax.iota(jnp.int32, VL)

    def zero_body(j, carry):
        z = lanes * 0
        for ref in (i00, i10, i01, i11):
            ref[pl.ds(j * VL, VL)] = z
        return carry

    lax.fori_loop(0, 2 * B // VL, zero_body, 0, unroll=False)

    def chunk_body(chunk, carry):
        base_pix = wid * P_PER_W + chunk * B
        pltpu.sync_copy(texc.at[pl.ds(base_pix * 2, 2 * B)], texc_v)

        def idx_body(j, carry):
            b16 = j * VL
            t2 = lanes * 2 + b16 * 2
            u = plsc.load_gather(texc_v, [t2])
            v = plsc.load_gather(texc_v, [t2 + 1])
            uu = u * float(W) - 0.5
            vv = v * float(H) - 0.5
            iut = uu.astype(jnp.int32)
            ivt = vv.astype(jnp.int32)
            ix0 = iut - (uu < iut.astype(jnp.float32)).astype(jnp.int32)
            iy0 = ivt - (vv < ivt.astype(jnp.float32)).astype(jnp.int32)
            fx = uu - ix0.astype(jnp.float32)
            fy = vv - iy0.astype(jnp.float32)
            ixw0 = ix0 & (W - 1)
            ixw1 = (ix0 + 1) & (W - 1)
            iyw0 = iy0 & (H - 1)
            iyw1 = (iy0 + 1) & (H - 1)
            xp0 = ((ixw0 >> 8) << 16) | (ixw0 & (PS - 1))
            xp1 = ((ixw1 >> 8) << 16) | (ixw1 & (PS - 1))
            yp0 = ((iyw0 >> 8) << 19) | ((iyw0 & (PS - 1)) << 8)
            yp1 = ((iyw1 >> 8) << 19) | ((iyw1 & (PS - 1)) << 8)
            slot = (lanes + b16) * 2
            plsc.store_scatter(i00, [slot], (yp0 | xp0) * 2)
            plsc.store_scatter(i10, [slot], (yp0 | xp1) * 2)
            plsc.store_scatter(i01, [slot], (yp1 | xp0) * 2)
            plsc.store_scatter(i11, [slot], (yp1 | xp1) * 2)
            fx_v[pl.ds(b16, VL)] = fx
            fy_v[pl.ds(b16, VL)] = fy
            return carry

        lax.fori_loop(0, NV, idx_body, 0, unroll=False)

        def fire(k, carry):
            for idx, rows in ((i00, r00), (i10, r10), (i01, r01), (i11, r11)):
                pltpu.make_async_copy(
                    table.at[idx.at[pl.ds(k * 2 * GB, 2 * GB)]],
                    rows.at[pl.ds(k * 2 * GB, 2 * GB)], sem).start()
            return carry

        lax.fori_loop(0, K, fire, 0, unroll=False)

        def drain(k, carry):
            for idx, rows in ((i00, r00), (i10, r10), (i01, r01), (i11, r11)):
                pltpu.make_async_copy(
                    table.at[idx.at[pl.ds(k * 2 * GB, 2 * GB)]],
                    rows.at[pl.ds(k * 2 * GB, 2 * GB)], sem).wait()
            return carry

        lax.fori_loop(0, K, drain, 0, unroll=False)

        def blend_body(j, carry):
            b16 = j * VL
            pix = lanes + b16
            row = pix + (b16 & ~(GB - 1))
            fx = fx_v[pl.ds(b16, VL)]
            fy = fy_v[pl.ds(b16, VL)]
            gx = 1.0 - fx
            gy = 1.0 - fy
            w00 = gx * gy
            w10 = fx * gy
            w01 = gx * fy
            w11 = fx * fy
            for c in range(C):
                cc = lanes * 0 + c
                c00 = plsc.load_gather(r00, [row, cc])
                c10 = plsc.load_gather(r10, [row, cc])
                c01 = plsc.load_gather(r01, [row, cc])
                c11 = plsc.load_gather(r11, [row, cc])
                o = c00 * w00 + c10 * w10 + c01 * w01 + c11 * w11
                plsc.store_scatter(out_v, [pix, cc], o)
            return carry

        lax.fori_loop(0, NV, blend_body, 0, unroll=False)

        pltpu.sync_copy(out_v, out.at[pl.ds(base_pix, B)])
        return carry

    lax.fori_loop(0, NCHUNK, chunk_body, 0, unroll=False)


@jax.jit
def _sample(table, texc_flat):
    mesh = plsc.VectorSubcoreMesh(core_axis_name="c", subcore_axis_name="s")
    f = pl.kernel(
        _body,
        out_type=jax.ShapeDtypeStruct((NPIX, C), jnp.float32),
        mesh=mesh,
        compiler_params=pltpu.CompilerParams(
            needs_layout_passes=False, use_tc_tiling_on_sc=False),
        scratch_types=[
            pltpu.VMEM((2 * B,), jnp.float32),
            pltpu.VMEM((2 * B,), jnp.int32),
            pltpu.VMEM((2 * B,), jnp.int32),
            pltpu.VMEM((2 * B,), jnp.int32),
            pltpu.VMEM((2 * B,), jnp.int32),
            pltpu.VMEM((B,), jnp.float32),
            pltpu.VMEM((B,), jnp.float32),
            pltpu.VMEM((2 * B, C), jnp.float32),
            pltpu.VMEM((2 * B, C), jnp.float32),
            pltpu.VMEM((2 * B, C), jnp.float32),
            pltpu.VMEM((2 * B, C), jnp.float32),
            pltpu.VMEM((B, C), jnp.float32),
            pltpu.SemaphoreType.DMA,
        ],
    )
    return f(table, texc_flat)


def kernel(pages, texc):
    table = pages.reshape(H * W, C)
    texc_flat = texc.reshape(-1)
    out = _sample(table, texc_flat)
    return out.reshape(1, OUT_H, OUT_W, C)


# speedup vs baseline: 3.6587x; 3.6559x over previous
import jax
import jax.numpy as jnp
from jax import lax
from jax.experimental import pallas as pl
from jax.experimental.pallas import tpu as pltpu
from jax.experimental.pallas import tpu_sc as plsc

H = 2048
W = 2048
C = 4
PS = 256
NPX = W // PS
OUT_H = 1024
OUT_W = 1024
NPIX = OUT_H * OUT_W

NC = 2
NS = 16
NW = NC * NS

M = W // 2
RW = 4 * 2 * C
P_PER_W = NPIX // NW
B = 1024
NCHUNK = P_PER_W // B
GB = 512
K = B // GB
VL = 16
NV = B // VL


def _body(table, texc, out, texc_v, idx_v, off_v, fx_v, fy_v, rows_v,
          out_v, sem):
    wid = lax.axis_index("s") * NC + lax.axis_index("c")
    lanes = lax.iota(jnp.int32, VL)

    def chunk_body(chunk, carry):
        base_pix = wid * P_PER_W + chunk * B
        pltpu.sync_copy(texc.at[pl.ds(base_pix * 2, 2 * B)], texc_v)

        def idx_body(j, carry):
            b16 = j * VL
            t2 = lanes * 2 + b16 * 2
            u = plsc.load_gather(texc_v, [t2])
            v = plsc.load_gather(texc_v, [t2 + 1])
            uu = u * float(W) - 0.5
            vv = v * float(H) - 0.5
            iut = uu.astype(jnp.int32)
            ivt = vv.astype(jnp.int32)
            ix0 = iut - (uu < iut.astype(jnp.float32)).astype(jnp.int32)
            iy0 = ivt - (vv < ivt.astype(jnp.float32)).astype(jnp.int32)
            fx = uu - ix0.astype(jnp.float32)
            fy = vv - iy0.astype(jnp.float32)
            ixw = ix0 & (W - 1)
            iyw = iy0 & (H - 1)
            rec = iyw * M + (ixw >> 1)
            idx_v[pl.ds(b16, VL)] = rec
            off_v[pl.ds(b16, VL)] = (ixw & 1) * C
            fx_v[pl.ds(b16, VL)] = fx
            fy_v[pl.ds(b16, VL)] = fy
            return carry

        lax.fori_loop(0, NV, idx_body, 0, unroll=False)

        def fire(k, carry):
            pltpu.make_async_copy(
                table.at[idx_v.at[pl.ds(k * GB, GB)]],
                rows_v.at[pl.ds(k * GB, GB)], sem).start()
            return carry

        lax.fori_loop(0, K, fire, 0, unroll=False)

        def drain(k, carry):
            pltpu.make_async_copy(
                table.at[idx_v.at[pl.ds(k * GB, GB)]],
                rows_v.at[pl.ds(k * GB, GB)], sem).wait()
            return carry

        lax.fori_loop(0, K, drain, 0, unroll=False)

        def blend_body(j, carry):
            b16 = j * VL
            pix = lanes + b16
            off = off_v[pl.ds(b16, VL)]
            fx = fx_v[pl.ds(b16, VL)]
            fy = fy_v[pl.ds(b16, VL)]
            gx = 1.0 - fx
            gy = 1.0 - fy
            w00 = gx * gy
            w10 = fx * gy
            w01 = gx * fy
            w11 = fx * fy
            for c in range(C):
                c00 = plsc.load_gather(rows_v, [pix, off + c])
                c10 = plsc.load_gather(rows_v, [pix, off + (C + c)])
                c01 = plsc.load_gather(rows_v, [pix, off + (4 * C + c)])
                c11 = plsc.load_gather(rows_v, [pix, off + (5 * C + c)])
                o = c00 * w00 + c10 * w10 + c01 * w01 + c11 * w11
                cc = lanes * 0 + c
                plsc.store_scatter(out_v, [pix, cc], o)
            return carry

        lax.fori_loop(0, NV, blend_body, 0, unroll=False)

        pltpu.sync_copy(out_v, out.at[pl.ds(base_pix, B)])
        return carry

    lax.fori_loop(0, NCHUNK, chunk_body, 0, unroll=False)


@jax.jit
def _sample(table, texc_flat):
    mesh = plsc.VectorSubcoreMesh(core_axis_name="c", subcore_axis_name="s")
    f = pl.kernel(
        _body,
        out_type=jax.ShapeDtypeStruct((NPIX, C), jnp.float32),
        mesh=mesh,
        compiler_params=pltpu.CompilerParams(
            needs_layout_passes=False, use_tc_tiling_on_sc=False),
        scratch_types=[
            pltpu.VMEM((2 * B,), jnp.float32),
            pltpu.VMEM((B,), jnp.int32),
            pltpu.VMEM((B,), jnp.int32),
            pltpu.VMEM((B,), jnp.float32),
            pltpu.VMEM((B,), jnp.float32),
            pltpu.VMEM((B, RW), jnp.float32),
            pltpu.VMEM((B, C), jnp.float32),
            pltpu.SemaphoreType.DMA,
        ],
    )
    return f(table, texc_flat)


def _build_overlap_table(pages):
    rowtex = (pages.reshape(NPX, NPX, PS, PS, C)
              .transpose(0, 2, 1, 3, 4)
              .reshape(H, W * C))
    rowtex_dn = jnp.roll(rowtex, -1, axis=0)
    a = rowtex.reshape(H, M, 2 * C)
    a2 = jnp.roll(rowtex, -2 * C, axis=1).reshape(H, M, 2 * C)
    b = rowtex_dn.reshape(H, M, 2 * C)
    b2 = jnp.roll(rowtex_dn, -2 * C, axis=1).reshape(H, M, 2 * C)
    return jnp.stack([a, a2, b, b2], axis=2).reshape(H * M, RW)


def kernel(pages, texc):
    table = _build_overlap_table(pages)
    texc_flat = texc.reshape(-1)
    out = _sample(table, texc_flat)
    return out.reshape(1, OUT_H, OUT_W, C)


# speedup vs baseline: 3.6848x; 1.0071x over previous
import jax
import jax.numpy as jnp
from jax import lax
from jax.experimental import pallas as pl
from jax.experimental.pallas import tpu as pltpu
from jax.experimental.pallas import tpu_sc as plsc

H = 2048
W = 2048
C = 4
PS = 256
NPX = W // PS
OUT_H = 1024
OUT_W = 1024
NPIX = OUT_H * OUT_W

NC = 2
NS = 16
NW = NC * NS

M = W // 2
RW = 4 * 2 * C
P_PER_W = NPIX // NW
B = 1024
NCHUNK = P_PER_W // B
GB = 512
K = B // GB
VL = 16
NV = B // VL


def _body(table, texc, out,
          texc_a, idx_a, off_a, fx_a, fy_a, rows_a, out_a, sem_a,
          texc_b, idx_b, off_b, fx_b, fy_b, rows_b, out_b, sem_b):
    wid = lax.axis_index("s") * NC + lax.axis_index("c")
    lanes = lax.iota(jnp.int32, VL)

    def idx_phase(chunk, texc_v, idx_v, off_v, fx_v, fy_v):
        base_pix = wid * P_PER_W + chunk * B
        pltpu.sync_copy(texc.at[pl.ds(base_pix * 2, 2 * B)], texc_v)

        def idx_body(j, carry):
            b16 = j * VL
            t2 = lanes * 2 + b16 * 2
            u = plsc.load_gather(texc_v, [t2])
            v = plsc.load_gather(texc_v, [t2 + 1])
            uu = u * float(W) - 0.5
            vv = v * float(H) - 0.5
            iut = uu.astype(jnp.int32)
            ivt = vv.astype(jnp.int32)
            ix0 = iut - (uu < iut.astype(jnp.float32)).astype(jnp.int32)
            iy0 = ivt - (vv < ivt.astype(jnp.float32)).astype(jnp.int32)
            fx = uu - ix0.astype(jnp.float32)
            fy = vv - iy0.astype(jnp.float32)
            ixw = ix0 & (W - 1)
            iyw = iy0 & (H - 1)
            rec = iyw * M + (ixw >> 1)
            idx_v[pl.ds(b16, VL)] = rec
            off_v[pl.ds(b16, VL)] = (ixw & 1) * C
            fx_v[pl.ds(b16, VL)] = fx
            fy_v[pl.ds(b16, VL)] = fy
            return carry

        lax.fori_loop(0, NV, idx_body, 0, unroll=False)

    def fire(idx_v, rows_v, sem):
        def fire_k(k, carry):
            pltpu.make_async_copy(
                table.at[idx_v.at[pl.ds(k * GB, GB)]],
                rows_v.at[pl.ds(k * GB, GB)], sem).start()
            return carry

        lax.fori_loop(0, K, fire_k, 0, unroll=False)

    def drain_blend(chunk, idx_v, off_v, fx_v, fy_v, rows_v, out_v, sem):
        base_pix = wid * P_PER_W + chunk * B

        def drain_k(k, carry):
            pltpu.make_async_copy(
                table.at[idx_v.at[pl.ds(k * GB, GB)]],
                rows_v.at[pl.ds(k * GB, GB)], sem).wait()
            return carry

        lax.fori_loop(0, K, drain_k, 0, unroll=False)

        def blend_body(j, carry):
            b16 = j * VL
            pix = lanes + b16
            off = off_v[pl.ds(b16, VL)]
            fx = fx_v[pl.ds(b16, VL)]
            fy = fy_v[pl.ds(b16, VL)]
            gx = 1.0 - fx
            gy = 1.0 - fy
            w00 = gx * gy
            w10 = fx * gy
            w01 = gx * fy
            w11 = fx * fy
            for c in range(C):
                c00 = plsc.load_gather(rows_v, [pix, off + c])
                c10 = plsc.load_gather(rows_v, [pix, off + (C + c)])
                c01 = plsc.load_gather(rows_v, [pix, off + (4 * C + c)])
                c11 = plsc.load_gather(rows_v, [pix, off + (5 * C + c)])
                o = c00 * w00 + c10 * w10 + c01 * w01 + c11 * w11
                cc = lanes * 0 + c
                plsc.store_scatter(out_v, [pix, cc], o)
            return carry

        lax.fori_loop(0, NV, blend_body, 0, unroll=False)
        pltpu.sync_copy(out_v, out.at[pl.ds(base_pix, B)])

    bufs_a = (texc_a, idx_a, off_a, fx_a, fy_a, rows_a, out_a, sem_a)
    bufs_b = (texc_b, idx_b, off_b, fx_b, fy_b, rows_b, out_b, sem_b)

    def pair_body(i, carry):
        ca = 2 * i
        cb = 2 * i + 1
        idx_phase(ca, texc_a, idx_a, off_a, fx_a, fy_a)
        fire(idx_a, rows_a, sem_a)
        idx_phase(cb, texc_b, idx_b, off_b, fx_b, fy_b)
        fire(idx_b, rows_b, sem_b)
        drain_blend(ca, idx_a, off_a, fx_a, fy_a, rows_a, out_a, sem_a)
        drain_blend(cb, idx_b, off_b, fx_b, fy_b, rows_b, out_b, sem_b)
        return carry

    lax.fori_loop(0, NCHUNK // 2, pair_body, 0, unroll=False)


@jax.jit
def _sample(table, texc_flat):
    mesh = plsc.VectorSubcoreMesh(core_axis_name="c", subcore_axis_name="s")
    f = pl.kernel(
        _body,
        out_type=jax.ShapeDtypeStruct((NPIX, C), jnp.float32),
        mesh=mesh,
        compiler_params=pltpu.CompilerParams(
            needs_layout_passes=False, use_tc_tiling_on_sc=False),
        scratch_types=[
            pltpu.VMEM((2 * B,), jnp.float32),
            pltpu.VMEM((B,), jnp.int32),
            pltpu.VMEM((B,), jnp.int32),
            pltpu.VMEM((B,), jnp.float32),
            pltpu.VMEM((B,), jnp.float32),
            pltpu.VMEM((B, RW), jnp.float32),
            pltpu.VMEM((B, C), jnp.float32),
            pltpu.SemaphoreType.DMA,
        ] * 2,
    )
    return f(table, texc_flat)


def _build_overlap_table(pages):
    rowtex = (pages.reshape(NPX, NPX, PS, PS, C)
              .transpose(0, 2, 1, 3, 4)
              .reshape(H, W * C))
    rowtex_dn = jnp.roll(rowtex, -1, axis=0)
    a = rowtex.reshape(H, M, 2 * C)
    a2 = jnp.roll(rowtex, -2 * C, axis=1).reshape(H, M, 2 * C)
    b = rowtex_dn.reshape(H, M, 2 * C)
    b2 = jnp.roll(rowtex_dn, -2 * C, axis=1).reshape(H, M, 2 * C)
    return jnp.stack([a, a2, b, b2], axis=2).reshape(H * M, RW)


def kernel(pages, texc):
    table = _build_overlap_table(pages)
    texc_flat = texc.reshape(-1)
    out = _sample(table, texc_flat)
    return out.reshape(1, OUT_H, OUT_W, C)


# speedup vs baseline: 3.6857x; 1.0002x over previous
import jax
import jax.numpy as jnp
from jax import lax
from jax.experimental import pallas as pl
from jax.experimental.pallas import tpu as pltpu
from jax.experimental.pallas import tpu_sc as plsc

H = 2048
W = 2048
C = 4
PS = 256
NPX = W // PS
OUT_H = 1024
OUT_W = 1024
NPIX = OUT_H * OUT_W

NC = 2
NS = 16
NW = NC * NS

M = W // 2
RW = 4 * 2 * C
P_PER_W = NPIX // NW
B = 1024
NCHUNK = P_PER_W // B
GB = 512
K = B // GB
VL = 16
NV = B // VL


def _body(table, texc, out,
          texc_a, idx_a, off_a, fx_a, fy_a, rows_a, out_a, sem_a,
          texc_b, idx_b, off_b, fx_b, fy_b, rows_b, out_b, sem_b):
    wid = lax.axis_index("s") * NC + lax.axis_index("c")
    lanes = lax.iota(jnp.int32, VL)

    def idx_phase(chunk, texc_v, idx_v, off_v, fx_v, fy_v):
        base_pix = wid * P_PER_W + chunk * B
        pltpu.sync_copy(texc.at[pl.ds(base_pix * 2, 2 * B)], texc_v)

        def idx_body(j, carry):
            b16 = j * VL
            t2 = lanes * 2 + b16 * 2
            u = plsc.load_gather(texc_v, [t2])
            v = plsc.load_gather(texc_v, [t2 + 1])
            uu = u * float(W) - 0.5
            vv = v * float(H) - 0.5
            iut = uu.astype(jnp.int32)
            ivt = vv.astype(jnp.int32)
            ix0 = iut - (uu < iut.astype(jnp.float32)).astype(jnp.int32)
            iy0 = ivt - (vv < ivt.astype(jnp.float32)).astype(jnp.int32)
            fx = uu - ix0.astype(jnp.float32)
            fy = vv - iy0.astype(jnp.float32)
            ixw = ix0 & (W - 1)
            iyw = iy0 & (H - 1)
            rec = iyw * M + (ixw >> 1)
            idx_v[pl.ds(b16, VL)] = rec
            off_v[pl.ds(b16, VL)] = (ixw & 1) * C
            fx_v[pl.ds(b16, VL)] = fx
            fy_v[pl.ds(b16, VL)] = fy
            return carry

        lax.fori_loop(0, NV, idx_body, 0, unroll=False)

    def fire(idx_v, rows_v, sem):
        def fire_k(k, carry):
            pltpu.make_async_copy(
                table.at[idx_v.at[pl.ds(k * GB, GB)]],
                rows_v.at[pl.ds(k * GB, GB)], sem).start()
            return carry

        lax.fori_loop(0, K, fire_k, 0, unroll=False)

    def drain_blend(chunk, idx_v, off_v, fx_v, fy_v, rows_v, out_v, sem):
        base_pix = wid * P_PER_W + chunk * B

        def drain_k(k, carry):
            pltpu.make_async_copy(
                table.at[idx_v.at[pl.ds(k * GB, GB)]],
                rows_v.at[pl.ds(k * GB, GB)], sem).wait()
            return carry

        lax.fori_loop(0, K, drain_k, 0, unroll=False)

        def blend_body(j, carry):
            b16 = j * VL
            pix = lanes + b16
            off = off_v[pl.ds(b16, VL)]
            fx = fx_v[pl.ds(b16, VL)]
            fy = fy_v[pl.ds(b16, VL)]
            gx = 1.0 - fx
            gy = 1.0 - fy
            w00 = gx * gy
            w10 = fx * gy
            w01 = gx * fy
            w11 = fx * fy
            for c in range(C):
                c00 = plsc.load_gather(rows_v, [pix, off + c])
                c10 = plsc.load_gather(rows_v, [pix, off + (C + c)])
                c01 = plsc.load_gather(rows_v, [pix, off + (4 * C + c)])
                c11 = plsc.load_gather(rows_v, [pix, off + (5 * C + c)])
                o = c00 * w00 + c10 * w10 + c01 * w01 + c11 * w11
                cc = lanes * 0 + c
                plsc.store_scatter(out_v, [pix, cc], o)
            return carry

        lax.fori_loop(0, NV, blend_body, 0, unroll=False)
        pltpu.sync_copy(out_v, out.at[pl.ds(base_pix, B)])

    def pair_body(i, carry):
        ca = 2 * i
        cb = 2 * i + 1
        idx_phase(ca, texc_a, idx_a, off_a, fx_a, fy_a)
        fire(idx_a, rows_a, sem_a)
        idx_phase(cb, texc_b, idx_b, off_b, fx_b, fy_b)
        fire(idx_b, rows_b, sem_b)
        drain_blend(ca, idx_a, off_a, fx_a, fy_a, rows_a, out_a, sem_a)
        drain_blend(cb, idx_b, off_b, fx_b, fy_b, rows_b, out_b, sem_b)
        return carry

    lax.fori_loop(0, NCHUNK // 2, pair_body, 0, unroll=False)


@jax.jit
def _sample(table, texc_flat):
    mesh = plsc.VectorSubcoreMesh(core_axis_name="c", subcore_axis_name="s")
    f = pl.kernel(
        _body,
        out_type=jax.ShapeDtypeStruct((NPIX, C), jnp.float32),
        mesh=mesh,
        compiler_params=pltpu.CompilerParams(
            needs_layout_passes=False, use_tc_tiling_on_sc=False),
        scratch_types=[
            pltpu.VMEM((2 * B,), jnp.float32),
            pltpu.VMEM((B,), jnp.int32),
            pltpu.VMEM((B,), jnp.int32),
            pltpu.VMEM((B,), jnp.float32),
            pltpu.VMEM((B,), jnp.float32),
            pltpu.VMEM((B, RW), jnp.float32),
            pltpu.VMEM((B, C), jnp.float32),
            pltpu.SemaphoreType.DMA,
        ] * 2,
    )
    return f(table, texc_flat)


def _build_overlap_table(pages):
    rowtex = (pages.reshape(NPX, NPX, PS, PS, C)
              .transpose(0, 2, 1, 3, 4)
              .reshape(H, W * C))
    rowtex_dn = jnp.roll(rowtex, -1, axis=0)
    a = rowtex.reshape(H, M, 2 * C)
    a2 = jnp.roll(rowtex, -2 * C, axis=1).reshape(H, M, 2 * C)
    b = rowtex_dn.reshape(H, M, 2 * C)
    b2 = jnp.roll(rowtex_dn, -2 * C, axis=1).reshape(H, M, 2 * C)
    return jnp.stack([a, a2, b, b2], axis=2).reshape(H * M, RW)


def kernel(pages, texc):
    table = _build_overlap_table(pages)
    texc_flat = texc.reshape(-1)
    out = _sample(table, texc_flat)
    return out.reshape(1, OUT_H, OUT_W, C)
